# baseline (speedup 1.0000x reference)
import functools

import jax
import jax.numpy as jnp
from jax import lax
from jax.experimental import pallas as pl
from jax.experimental.pallas import tpu as pltpu
from jax.experimental.pallas import tpu_sc as plsc

N = 512
E = 32768
IN_FEATS = 256
HID = 64
T = 3
OUT_DIM = 1024

EDGE_BLK = 2048
NUM_EDGE_BLKS = E // EDGE_BLK
FC2_BLK = 2048
NUM_FC2_BLKS = (N * HID * 2) // FC2_BLK


def _leaky(x):
    return jnp.where(x >= 0, x, 0.01 * x)


NSC = 2
NSUB = 16
EPW = E // (NSC * NSUB)
RTOT = T * N * N
ROWS_PW = RTOT // NSUB


def _sc_build_b_body(dst_hbm, src_hbm, et_hbm, zeros_hbm, ones_hbm,
                     out_hbm, dstb, srcb, etb, idxb, onesv, a_sh):
    c = lax.axis_index("c")
    s = lax.axis_index("s")
    base_e = (c * NSUB + s) * EPW
    pltpu.sync_copy(dst_hbm.at[pl.ds(base_e, EPW)], dstb)
    pltpu.sync_copy(src_hbm.at[pl.ds(base_e, EPW)], srcb)
    pltpu.sync_copy(et_hbm.at[pl.ds(base_e, EPW)], etb)
    pltpu.sync_copy(ones_hbm, onesv)
    pltpu.sync_copy(zeros_hbm.at[pl.ds(s * ROWS_PW, ROWS_PW)],
                    a_sh.at[pl.ds(s * ROWS_PW, ROWS_PW)])
    for k in range(EPW // 16):
        d16 = dstb[pl.ds(k * 16, 16)]
        s16 = srcb[pl.ds(k * 16, 16)]
        e16 = etb[pl.ds(k * 16, 16)]
        idxb[k // 8, pl.ds((k % 8) * 16, 16)] = e16 * (N * N) + d16 * N + s16
    plsc.subcore_barrier()
    for j in range(8):
        pltpu.sync_copy(onesv, a_sh.at[idxb.at[j]], add=True)
    plsc.subcore_barrier()
    pltpu.sync_copy(a_sh.at[pl.ds(s * ROWS_PW, ROWS_PW)],
                    out_hbm.at[c, pl.ds(s * ROWS_PW, ROWS_PW)])


def _build_b_sc(edge_index, edge_type):
    kern = pl.kernel(
        _sc_build_b_body,
        mesh=plsc.VectorSubcoreMesh(core_axis_name="c", subcore_axis_name="s"),
        out_type=jax.ShapeDtypeStruct((NSC, RTOT), jnp.float32),
        scratch_types=[
            pltpu.VMEM((EPW,), jnp.int32),
            pltpu.VMEM((EPW,), jnp.int32),
            pltpu.VMEM((EPW,), jnp.int32),
            pltpu.VMEM((8, 128), jnp.int32),
            pltpu.VMEM((128,), jnp.float32),
            pltpu.VMEM_SHARED((RTOT,), jnp.float32),
        ],
    )
    zeros = jnp.zeros((RTOT,), jnp.float32)
    ones = jnp.ones((128,), jnp.float32)
    return kern(edge_index[1], edge_index[0], edge_type, zeros, ones)


def _build_a_kernel(et_ref, dst_ref, src_ref, a_ref):
    i = pl.program_id(0)

    @pl.when(i == 0)
    def _():
        a_ref[...] = jnp.zeros_like(a_ref)

    et = et_ref[0, 0, :]
    dst = dst_ref[0, 0, :]
    src = src_ref[0, 0, :]
    col = et * N + src
    r_iota = jax.lax.broadcasted_iota(jnp.int32, (N, EDGE_BLK), 0)
    u = (r_iota == dst[None, :]).astype(jnp.bfloat16)
    c_iota = jax.lax.broadcasted_iota(jnp.int32, (EDGE_BLK, T * N), 1)
    v = (col[:, None] == c_iota).astype(jnp.bfloat16)
    a_ref[...] += jnp.dot(u, v, preferred_element_type=jnp.float32)


def _build_a(edge_index, edge_type):
    et = edge_type.reshape(NUM_EDGE_BLKS, 1, EDGE_BLK)
    src = edge_index[0].reshape(NUM_EDGE_BLKS, 1, EDGE_BLK)
    dst = edge_index[1].reshape(NUM_EDGE_BLKS, 1, EDGE_BLK)
    blk = pl.BlockSpec((1, 1, EDGE_BLK), lambda i: (i, 0, 0))
    return pl.pallas_call(
        _build_a_kernel,
        grid=(NUM_EDGE_BLKS,),
        in_specs=[blk, blk, blk],
        out_specs=pl.BlockSpec((N, T * N), lambda i: (0, 0)),
        out_shape=jax.ShapeDtypeStruct((N, T * N), jnp.float32),
    )(et, dst, src)


def _layers_kernel(x_ref, a_ref, we0, ws0, wk0, wg0, bg0, we1, ws1, wk1, wg1,
                   bg1, we2, ws2, wk2, wg2, bg2, hf_ref):
    def dot(a, b):
        return jnp.dot(a, b, preferred_element_type=jnp.float32)

    def dot_exact(a_bf16, x):
        acc = jnp.zeros((a_bf16.shape[0], x.shape[1]), jnp.float32)
        r = x
        for _ in range(3):
            c = r.astype(jnp.bfloat16)
            r = r - c.astype(jnp.float32)
            acc += jnp.dot(a_bf16, c, preferred_element_type=jnp.float32)
        return acc

    asum = a_ref[0] + a_ref[1]
    a_bf16 = jnp.concatenate([asum[t] for t in range(T)],
                             axis=1).astype(jnp.bfloat16)

    def layer(h, we, ws, wk, wg, bg, fin):
        hw = jnp.concatenate([dot(h, we[t]) for t in range(T)], axis=0)
        agg = dot_exact(a_bf16, hw)
        u = agg + dot(h, ws[...])
        g = jax.nn.sigmoid(dot(h, wg[:fin, :]) + dot(u, wg[fin:, :])
                           + bg[...][None, :])
        return g * _leaky(u) + (1.0 - g) * dot(h, wk[...])

    x = x_ref[...]
    h = layer(x, we0, ws0, wk0, wg0, bg0, IN_FEATS)
    h = layer(h, we1, ws1, wk1, wg1, bg1, HID)
    h = layer(h, we2, ws2, wk2, wg2, bg2, HID)
    hf_ref[...] = h


def _run_layers(x, a, args):
    return pl.pallas_call(
        _layers_kernel,
        out_shape=jax.ShapeDtypeStruct((N, HID), jnp.float32),
    )(x, a, *args)


def _fc1_kernel(x_ref, fc1w_ref, fc1b_ref, feat_ref):
    feat_ref[...] = _leaky(
        jnp.dot(x_ref[...], fc1w_ref[...],
                preferred_element_type=jnp.float32) + fc1b_ref[...])


def _run_fc1(x, fc1_W, fc1_b):
    return pl.pallas_call(
        _fc1_kernel,
        out_shape=jax.ShapeDtypeStruct((N, HID), jnp.float32),
    )(x, fc1_W, fc1_b.reshape(1, HID))


NBLK = 16


def _fc2_feat_kernel(f_ref, w_ref, out_ref):
    i = pl.program_id(0)

    @pl.when(i == 0)
    def _():
        out_ref[...] = jnp.zeros_like(out_ref)

    out_ref[...] += jnp.dot(
        f_ref[...], w_ref[...].reshape(NBLK * HID, OUT_DIM),
        preferred_element_type=jnp.float32)


def _run_fc2_feat(featflat, w3):
    return pl.pallas_call(
        _fc2_feat_kernel,
        grid=(N // NBLK,),
        in_specs=[
            pl.BlockSpec((1, NBLK * HID), lambda i: (0, i)),
            pl.BlockSpec((NBLK, HID, OUT_DIM), lambda i: (i, 1, 0)),
        ],
        out_specs=pl.BlockSpec((1, OUT_DIM), lambda i: (0, 0)),
        out_shape=jax.ShapeDtypeStruct((1, OUT_DIM), jnp.float32),
    )(featflat, w3)


def _fc2_h_kernel(h_ref, w_ref, p_ref, b_ref, out_ref):
    i = pl.program_id(0)

    @pl.when(i == 0)
    def _():
        out_ref[...] = jnp.zeros_like(out_ref)

    out_ref[...] += jnp.dot(
        h_ref[...],
        w_ref[...].reshape(NBLK * HID, OUT_DIM),
        preferred_element_type=jnp.float32)

    @pl.when(i == N // NBLK - 1)
    def _():
        out_ref[...] = _leaky(out_ref[...] + p_ref[...] + b_ref[...])


def _run_fc2_h(hflat, w3, partial, b):
    return pl.pallas_call(
        _fc2_h_kernel,
        grid=(N // NBLK,),
        in_specs=[
            pl.BlockSpec((1, NBLK * HID), lambda i: (0, i)),
            pl.BlockSpec((NBLK, HID, OUT_DIM), lambda i: (i, 0, 0)),
            pl.BlockSpec((1, OUT_DIM), lambda i: (0, 0)),
            pl.BlockSpec((1, OUT_DIM), lambda i: (0, 0)),
        ],
        out_specs=pl.BlockSpec((1, OUT_DIM), lambda i: (0, 0)),
        out_shape=jax.ShapeDtypeStruct((1, OUT_DIM), jnp.float32),
    )(hflat, w3, partial, b.reshape(1, OUT_DIM))


def kernel(x, edge_index, edge_type, W_et0, W_self0, W_skip0, W_gate0,
           b_gate0, W_et1, W_self1, W_skip1, W_gate1, b_gate1, W_et2,
           W_self2, W_skip2, W_gate2, b_gate2, fc1_W, fc1_b, fc2_W, fc2_b):
    w3 = fc2_W.reshape(N, 2 * HID, OUT_DIM)
    feat = _run_fc1(x, fc1_W, fc1_b)
    partial = _run_fc2_feat(feat.reshape(1, N * HID), w3)
    b_parts = _build_b_sc(edge_index, edge_type)
    a = b_parts.reshape(NSC, T, N, N)
    h = _run_layers(x, a, (W_et0, W_self0, W_skip0, W_gate0, b_gate0,
                           W_et1, W_self1, W_skip1, W_gate1, b_gate1,
                           W_et2, W_self2, W_skip2, W_gate2, b_gate2))
    return _run_fc2_h(h.reshape(1, N * HID), w3, partial, fc2_b)


# speedup vs baseline: 1.0658x; 1.0658x over previous
import functools

import jax
import jax.numpy as jnp
from jax import lax
from jax.experimental import pallas as pl
from jax.experimental.pallas import tpu as pltpu
from jax.experimental.pallas import tpu_sc as plsc

N = 512
E = 32768
IN_FEATS = 256
HID = 64
T = 3
OUT_DIM = 1024

EDGE_BLK = 2048
NUM_EDGE_BLKS = E // EDGE_BLK
FC2_BLK = 2048
NUM_FC2_BLKS = (N * HID * 2) // FC2_BLK


def _leaky(x):
    return jnp.where(x >= 0, x, 0.01 * x)


NSC = 2
NSUB = 16
EPW = E // (NSC * NSUB)
RTOT = T * N * N
ROWS_PW = RTOT // NSUB


def _sc_build_b_body(dst_hbm, src_hbm, et_hbm, zeros_hbm, ones_hbm,
                     out_hbm, dstb, srcb, etb, idxb, onesv, a_sh):
    c = lax.axis_index("c")
    s = lax.axis_index("s")
    base_e = (c * NSUB + s) * EPW
    pltpu.sync_copy(dst_hbm.at[pl.ds(base_e, EPW)], dstb)
    pltpu.sync_copy(src_hbm.at[pl.ds(base_e, EPW)], srcb)
    pltpu.sync_copy(et_hbm.at[pl.ds(base_e, EPW)], etb)
    pltpu.sync_copy(ones_hbm, onesv)
    pltpu.sync_copy(zeros_hbm.at[pl.ds(s * ROWS_PW, ROWS_PW)],
                    a_sh.at[pl.ds(s * ROWS_PW, ROWS_PW)])
    for k in range(EPW // 16):
        d16 = dstb[pl.ds(k * 16, 16)]
        s16 = srcb[pl.ds(k * 16, 16)]
        e16 = etb[pl.ds(k * 16, 16)]
        idxb[k // 8, pl.ds((k % 8) * 16, 16)] = e16 * (N * N) + d16 * N + s16
    plsc.subcore_barrier()
    for j in range(8):
        pltpu.sync_copy(onesv, a_sh.at[idxb.at[j]], add=True)
    plsc.subcore_barrier()
    pltpu.sync_copy(a_sh.at[pl.ds(s * ROWS_PW, ROWS_PW)],
                    out_hbm.at[c, pl.ds(s * ROWS_PW, ROWS_PW)])


def _build_b_sc(edge_index, edge_type):
    kern = pl.kernel(
        _sc_build_b_body,
        mesh=plsc.VectorSubcoreMesh(core_axis_name="c", subcore_axis_name="s"),
        out_type=jax.ShapeDtypeStruct((NSC, RTOT), jnp.float32),
        scratch_types=[
            pltpu.VMEM((EPW,), jnp.int32),
            pltpu.VMEM((EPW,), jnp.int32),
            pltpu.VMEM((EPW,), jnp.int32),
            pltpu.VMEM((8, 128), jnp.int32),
            pltpu.VMEM((128,), jnp.float32),
            pltpu.VMEM_SHARED((RTOT,), jnp.float32),
        ],
    )
    zeros = jnp.zeros((RTOT,), jnp.float32)
    ones = jnp.ones((128,), jnp.float32)
    return kern(edge_index[1], edge_index[0], edge_type, zeros, ones)


def _build_a_kernel(et_ref, dst_ref, src_ref, a_ref):
    i = pl.program_id(0)

    @pl.when(i == 0)
    def _():
        a_ref[...] = jnp.zeros_like(a_ref)

    et = et_ref[0, 0, :]
    dst = dst_ref[0, 0, :]
    src = src_ref[0, 0, :]
    col = et * N + src
    r_iota = jax.lax.broadcasted_iota(jnp.int32, (N, EDGE_BLK), 0)
    u = (r_iota == dst[None, :]).astype(jnp.bfloat16)
    c_iota = jax.lax.broadcasted_iota(jnp.int32, (EDGE_BLK, T * N), 1)
    v = (col[:, None] == c_iota).astype(jnp.bfloat16)
    a_ref[...] += jnp.dot(u, v, preferred_element_type=jnp.float32)


def _build_a(edge_index, edge_type):
    et = edge_type.reshape(NUM_EDGE_BLKS, 1, EDGE_BLK)
    src = edge_index[0].reshape(NUM_EDGE_BLKS, 1, EDGE_BLK)
    dst = edge_index[1].reshape(NUM_EDGE_BLKS, 1, EDGE_BLK)
    blk = pl.BlockSpec((1, 1, EDGE_BLK), lambda i: (i, 0, 0))
    return pl.pallas_call(
        _build_a_kernel,
        grid=(NUM_EDGE_BLKS,),
        in_specs=[blk, blk, blk],
        out_specs=pl.BlockSpec((N, T * N), lambda i: (0, 0)),
        out_shape=jax.ShapeDtypeStruct((N, T * N), jnp.float32),
    )(et, dst, src)


def _layers_kernel(x_ref, a_ref, we0, ws0, wk0, wg0, bg0, we1, ws1, wk1, wg1,
                   bg1, we2, ws2, wk2, wg2, bg2, hf_ref):
    def dot(a, b):
        return jnp.dot(a, b, preferred_element_type=jnp.float32)

    def dot_exact(a_bf16, x):
        acc = jnp.zeros((a_bf16.shape[0], x.shape[1]), jnp.float32)
        r = x
        for _ in range(3):
            c = r.astype(jnp.bfloat16)
            r = r - c.astype(jnp.float32)
            acc += jnp.dot(a_bf16, c, preferred_element_type=jnp.float32)
        return acc

    asum = a_ref[0] + a_ref[1]
    a_bf16 = jnp.concatenate([asum[t] for t in range(T)],
                             axis=1).astype(jnp.bfloat16)

    def layer(h, we, ws, wk, wg, bg, fin):
        hw = jnp.concatenate([dot(h, we[t]) for t in range(T)], axis=0)
        agg = dot_exact(a_bf16, hw)
        u = agg + dot(h, ws[...])
        g = jax.nn.sigmoid(dot(h, wg[:fin, :]) + dot(u, wg[fin:, :])
                           + bg[...][None, :])
        return g * _leaky(u) + (1.0 - g) * dot(h, wk[...])

    x = x_ref[...]
    h = layer(x, we0, ws0, wk0, wg0, bg0, IN_FEATS)
    h = layer(h, we1, ws1, wk1, wg1, bg1, HID)
    h = layer(h, we2, ws2, wk2, wg2, bg2, HID)
    hf_ref[...] = h


def _run_layers(x, a, args):
    return pl.pallas_call(
        _layers_kernel,
        out_shape=jax.ShapeDtypeStruct((N, HID), jnp.float32),
    )(x, a, *args)


def _fc1_kernel(x_ref, fc1w_ref, fc1b_ref, feat_ref):
    feat_ref[...] = _leaky(
        jnp.dot(x_ref[...], fc1w_ref[...],
                preferred_element_type=jnp.float32) + fc1b_ref[...])


def _run_fc1(x, fc1_W, fc1_b):
    return pl.pallas_call(
        _fc1_kernel,
        out_shape=jax.ShapeDtypeStruct((N, HID), jnp.float32),
    )(x, fc1_W, fc1_b.reshape(1, HID))


NBLK = 32


def _fc2_feat_kernel(f_ref, w_ref, out_ref):
    i = pl.program_id(0)

    @pl.when(i == 0)
    def _():
        out_ref[...] = jnp.zeros_like(out_ref)

    out_ref[...] += jnp.dot(
        f_ref[...], w_ref[...].reshape(NBLK * HID, OUT_DIM),
        preferred_element_type=jnp.float32)


def _run_fc2_feat(featflat, w3):
    return pl.pallas_call(
        _fc2_feat_kernel,
        grid=(N // NBLK,),
        in_specs=[
            pl.BlockSpec((1, NBLK * HID), lambda i: (0, i)),
            pl.BlockSpec((NBLK, HID, OUT_DIM), lambda i: (i, 1, 0)),
        ],
        out_specs=pl.BlockSpec((1, OUT_DIM), lambda i: (0, 0)),
        out_shape=jax.ShapeDtypeStruct((1, OUT_DIM), jnp.float32),
    )(featflat, w3)


def _fc2_h_kernel(h_ref, w_ref, p_ref, b_ref, out_ref):
    i = pl.program_id(0)

    @pl.when(i == 0)
    def _():
        out_ref[...] = jnp.zeros_like(out_ref)

    out_ref[...] += jnp.dot(
        h_ref[...],
        w_ref[...].reshape(NBLK * HID, OUT_DIM),
        preferred_element_type=jnp.float32)

    @pl.when(i == N // NBLK - 1)
    def _():
        out_ref[...] = _leaky(out_ref[...] + p_ref[...] + b_ref[...])


def _run_fc2_h(hflat, w3, partial, b):
    return pl.pallas_call(
        _fc2_h_kernel,
        grid=(N // NBLK,),
        in_specs=[
            pl.BlockSpec((1, NBLK * HID), lambda i: (0, i)),
            pl.BlockSpec((NBLK, HID, OUT_DIM), lambda i: (i, 0, 0)),
            pl.BlockSpec((1, OUT_DIM), lambda i: (0, 0)),
            pl.BlockSpec((1, OUT_DIM), lambda i: (0, 0)),
        ],
        out_specs=pl.BlockSpec((1, OUT_DIM), lambda i: (0, 0)),
        out_shape=jax.ShapeDtypeStruct((1, OUT_DIM), jnp.float32),
    )(hflat, w3, partial, b.reshape(1, OUT_DIM))


def kernel(x, edge_index, edge_type, W_et0, W_self0, W_skip0, W_gate0,
           b_gate0, W_et1, W_self1, W_skip1, W_gate1, b_gate1, W_et2,
           W_self2, W_skip2, W_gate2, b_gate2, fc1_W, fc1_b, fc2_W, fc2_b):
    w3 = fc2_W.reshape(N, 2 * HID, OUT_DIM)
    feat = _run_fc1(x, fc1_W, fc1_b)
    partial = _run_fc2_feat(feat.reshape(1, N * HID), w3)
    b_parts = _build_b_sc(edge_index, edge_type)
    a = b_parts.reshape(NSC, T, N, N)
    h = _run_layers(x, a, (W_et0, W_self0, W_skip0, W_gate0, b_gate0,
                           W_et1, W_self1, W_skip1, W_gate1, b_gate1,
                           W_et2, W_self2, W_skip2, W_gate2, b_gate2))
    return _run_fc2_h(h.reshape(1, N * HID), w3, partial, fc2_b)


# speedup vs baseline: 1.1020x; 1.0340x over previous
import functools

import jax
import jax.numpy as jnp
from jax import lax
from jax.experimental import pallas as pl
from jax.experimental.pallas import tpu as pltpu
from jax.experimental.pallas import tpu_sc as plsc

N = 512
E = 32768
IN_FEATS = 256
HID = 64
T = 3
OUT_DIM = 1024

EDGE_BLK = 2048
NUM_EDGE_BLKS = E // EDGE_BLK
FC2_BLK = 2048
NUM_FC2_BLKS = (N * HID * 2) // FC2_BLK


def _leaky(x):
    return jnp.where(x >= 0, x, 0.01 * x)


NSC = 2
NSUB = 16
EPW = E // (NSC * NSUB)
RTOT = T * N * N
ROWS_PW = RTOT // NSUB


def _sc_build_b_body(dst_hbm, src_hbm, et_hbm, zeros_hbm, ones_hbm,
                     out_hbm, dstb, srcb, etb, idxb, onesv, a_sh):
    c = lax.axis_index("c")
    s = lax.axis_index("s")
    base_e = (c * NSUB + s) * EPW
    pltpu.sync_copy(dst_hbm.at[pl.ds(base_e, EPW)], dstb)
    pltpu.sync_copy(src_hbm.at[pl.ds(base_e, EPW)], srcb)
    pltpu.sync_copy(et_hbm.at[pl.ds(base_e, EPW)], etb)
    pltpu.sync_copy(ones_hbm, onesv)
    pltpu.sync_copy(zeros_hbm.at[pl.ds(s * ROWS_PW, ROWS_PW)],
                    a_sh.at[pl.ds(s * ROWS_PW, ROWS_PW)])
    for k in range(EPW // 16):
        d16 = dstb[pl.ds(k * 16, 16)]
        s16 = srcb[pl.ds(k * 16, 16)]
        e16 = etb[pl.ds(k * 16, 16)]
        idxb[k // 8, pl.ds((k % 8) * 16, 16)] = e16 * (N * N) + d16 * N + s16
    plsc.subcore_barrier()
    for j in range(8):
        pltpu.sync_copy(onesv, a_sh.at[idxb.at[j]], add=True)
    plsc.subcore_barrier()
    pltpu.sync_copy(a_sh.at[pl.ds(s * ROWS_PW, ROWS_PW)],
                    out_hbm.at[c, pl.ds(s * ROWS_PW, ROWS_PW)])


def _build_b_sc(edge_index, edge_type):
    kern = pl.kernel(
        _sc_build_b_body,
        mesh=plsc.VectorSubcoreMesh(core_axis_name="c", subcore_axis_name="s"),
        out_type=jax.ShapeDtypeStruct((NSC, RTOT), jnp.float32),
        scratch_types=[
            pltpu.VMEM((EPW,), jnp.int32),
            pltpu.VMEM((EPW,), jnp.int32),
            pltpu.VMEM((EPW,), jnp.int32),
            pltpu.VMEM((8, 128), jnp.int32),
            pltpu.VMEM((128,), jnp.float32),
            pltpu.VMEM_SHARED((RTOT,), jnp.float32),
        ],
    )
    zeros = jnp.zeros((RTOT,), jnp.float32)
    ones = jnp.ones((128,), jnp.float32)
    return kern(edge_index[1], edge_index[0], edge_type, zeros, ones)


def _build_a_kernel(et_ref, dst_ref, src_ref, a_ref):
    i = pl.program_id(0)

    @pl.when(i == 0)
    def _():
        a_ref[...] = jnp.zeros_like(a_ref)

    et = et_ref[0, 0, :]
    dst = dst_ref[0, 0, :]
    src = src_ref[0, 0, :]
    col = et * N + src
    r_iota = jax.lax.broadcasted_iota(jnp.int32, (N, EDGE_BLK), 0)
    u = (r_iota == dst[None, :]).astype(jnp.bfloat16)
    c_iota = jax.lax.broadcasted_iota(jnp.int32, (EDGE_BLK, T * N), 1)
    v = (col[:, None] == c_iota).astype(jnp.bfloat16)
    a_ref[...] += jnp.dot(u, v, preferred_element_type=jnp.float32)


def _build_a(edge_index, edge_type):
    et = edge_type.reshape(NUM_EDGE_BLKS, 1, EDGE_BLK)
    src = edge_index[0].reshape(NUM_EDGE_BLKS, 1, EDGE_BLK)
    dst = edge_index[1].reshape(NUM_EDGE_BLKS, 1, EDGE_BLK)
    blk = pl.BlockSpec((1, 1, EDGE_BLK), lambda i: (i, 0, 0))
    return pl.pallas_call(
        _build_a_kernel,
        grid=(NUM_EDGE_BLKS,),
        in_specs=[blk, blk, blk],
        out_specs=pl.BlockSpec((N, T * N), lambda i: (0, 0)),
        out_shape=jax.ShapeDtypeStruct((N, T * N), jnp.float32),
    )(et, dst, src)


def _layers_kernel(x_ref, a_ref, we0, ws0, wk0, wg0, bg0, we1, ws1, wk1, wg1,
                   bg1, we2, ws2, wk2, wg2, bg2, hf_ref):
    def dot(a, b):
        return jnp.dot(a, b, preferred_element_type=jnp.float32)

    def dot_exact(a_bf16, x):
        acc = jnp.zeros((a_bf16.shape[0], x.shape[1]), jnp.float32)
        r = x
        for _ in range(3):
            c = r.astype(jnp.bfloat16)
            r = r - c.astype(jnp.float32)
            acc += jnp.dot(a_bf16, c, preferred_element_type=jnp.float32)
        return acc

    asum = a_ref[0] + a_ref[1]
    a_bf16 = jnp.concatenate([asum[t] for t in range(T)],
                             axis=1).astype(jnp.bfloat16)

    def layer(h, we, ws, wk, wg, bg, fin):
        hw = jnp.concatenate([dot(h, we[t]) for t in range(T)], axis=0)
        agg = dot_exact(a_bf16, hw)
        u = agg + dot(h, ws[...])
        g = jax.nn.sigmoid(dot(h, wg[:fin, :]) + dot(u, wg[fin:, :])
                           + bg[...][None, :])
        return g * _leaky(u) + (1.0 - g) * dot(h, wk[...])

    x = x_ref[...]
    h = layer(x, we0, ws0, wk0, wg0, bg0, IN_FEATS)
    h = layer(h, we1, ws1, wk1, wg1, bg1, HID)
    h = layer(h, we2, ws2, wk2, wg2, bg2, HID)
    hf_ref[...] = h


def _run_layers(x, a, args):
    return pl.pallas_call(
        _layers_kernel,
        out_shape=jax.ShapeDtypeStruct((N, HID), jnp.float32),
    )(x, a, *args)


def _fc1_kernel(x_ref, fc1w_ref, fc1b_ref, feat_ref):
    feat_ref[...] = _leaky(
        jnp.dot(x_ref[...], fc1w_ref[...],
                preferred_element_type=jnp.float32) + fc1b_ref[...])


def _run_fc1(x, fc1_W, fc1_b):
    return pl.pallas_call(
        _fc1_kernel,
        out_shape=jax.ShapeDtypeStruct((N, HID), jnp.float32),
    )(x, fc1_W, fc1_b.reshape(1, HID))


def _fc2_kernel(flat_ref, w_ref, b_ref, out_ref):
    i = pl.program_id(0)

    @pl.when(i == 0)
    def _():
        out_ref[...] = jnp.zeros_like(out_ref)

    out_ref[...] += jnp.dot(flat_ref[...], w_ref[...],
                            preferred_element_type=jnp.float32)

    @pl.when(i == NUM_FC2_BLKS - 1)
    def _():
        out_ref[...] = _leaky(out_ref[...] + b_ref[...])


def _run_fc2(flat, w, b):
    return pl.pallas_call(
        _fc2_kernel,
        grid=(NUM_FC2_BLKS,),
        in_specs=[
            pl.BlockSpec((1, FC2_BLK), lambda i: (0, i)),
            pl.BlockSpec((FC2_BLK, OUT_DIM), lambda i: (i, 0)),
            pl.BlockSpec((1, OUT_DIM), lambda i: (0, 0)),
        ],
        out_specs=pl.BlockSpec((1, OUT_DIM), lambda i: (0, 0)),
        out_shape=jax.ShapeDtypeStruct((1, OUT_DIM), jnp.float32),
    )(flat, w, b)


def kernel(x, edge_index, edge_type, W_et0, W_self0, W_skip0, W_gate0,
           b_gate0, W_et1, W_self1, W_skip1, W_gate1, b_gate1, W_et2,
           W_self2, W_skip2, W_gate2, b_gate2, fc1_W, fc1_b, fc2_W, fc2_b):
    feat = _run_fc1(x, fc1_W, fc1_b)
    b_parts = _build_b_sc(edge_index, edge_type)
    a = b_parts.reshape(NSC, T, N, N)
    h = _run_layers(x, a, (W_et0, W_self0, W_skip0, W_gate0, b_gate0,
                           W_et1, W_self1, W_skip1, W_gate1, b_gate1,
                           W_et2, W_self2, W_skip2, W_gate2, b_gate2))
    flat = jnp.concatenate([h, feat], axis=1).reshape(1, N * 2 * HID)
    return _run_fc2(flat, fc2_W, fc2_b.reshape(1, OUT_DIM))


# speedup vs baseline: 1.1167x; 1.0133x over previous
import jax
import jax.numpy as jnp
from jax import lax
from jax.experimental import pallas as pl
from jax.experimental.pallas import tpu as pltpu
from jax.experimental.pallas import tpu_sc as plsc

N = 512
E = 32768
IN_FEATS = 256
HID = 64
T = 3
OUT_DIM = 1024

FC2_BLK = 2048
NUM_FC2_BLKS = (N * HID * 2) // FC2_BLK


def _leaky(x):
    return jnp.where(x >= 0, x, 0.01 * x)


NSC = 2
NSUB = 16
EPW = E // (NSC * NSUB)
RTOT = T * N * N
ROWS_PW = RTOT // NSUB


def _sc_build_b_body(dst_hbm, src_hbm, et_hbm, zeros_hbm, ones_hbm,
                     out_hbm, dstb, srcb, etb, idxb, onesv, a_sh):
    c = lax.axis_index("c")
    s = lax.axis_index("s")
    base_e = (c * NSUB + s) * EPW
    pltpu.sync_copy(dst_hbm.at[pl.ds(base_e, EPW)], dstb)
    pltpu.sync_copy(src_hbm.at[pl.ds(base_e, EPW)], srcb)
    pltpu.sync_copy(et_hbm.at[pl.ds(base_e, EPW)], etb)
    pltpu.sync_copy(ones_hbm, onesv)
    pltpu.sync_copy(zeros_hbm.at[pl.ds(s * ROWS_PW, ROWS_PW)],
                    a_sh.at[pl.ds(s * ROWS_PW, ROWS_PW)])
    for k in range(EPW // 16):
        d16 = dstb[pl.ds(k * 16, 16)]
        s16 = srcb[pl.ds(k * 16, 16)]
        e16 = etb[pl.ds(k * 16, 16)]
        idxb[k // 8, pl.ds((k % 8) * 16, 16)] = e16 * (N * N) + d16 * N + s16
    plsc.subcore_barrier()
    for j in range(8):
        pltpu.sync_copy(onesv, a_sh.at[idxb.at[j]], add=True)
    plsc.subcore_barrier()
    pltpu.sync_copy(a_sh.at[pl.ds(s * ROWS_PW, ROWS_PW)],
                    out_hbm.at[c, pl.ds(s * ROWS_PW, ROWS_PW)])


def _build_b_sc(edge_index, edge_type):
    kern = pl.kernel(
        _sc_build_b_body,
        mesh=plsc.VectorSubcoreMesh(core_axis_name="c", subcore_axis_name="s"),
        out_type=jax.ShapeDtypeStruct((NSC, RTOT), jnp.float32),
        scratch_types=[
            pltpu.VMEM((EPW,), jnp.int32),
            pltpu.VMEM((EPW,), jnp.int32),
            pltpu.VMEM((EPW,), jnp.int32),
            pltpu.VMEM((8, 128), jnp.int32),
            pltpu.VMEM((128,), jnp.float32),
            pltpu.VMEM_SHARED((RTOT,), jnp.float32),
        ],
    )
    zeros = jnp.zeros((RTOT,), jnp.float32)
    ones = jnp.ones((128,), jnp.float32)
    return kern(edge_index[1], edge_index[0], edge_type, zeros, ones)


def _layers_kernel(x_ref, a_ref, we0, ws0, wk0, wg0, bg0, we1, ws1, wk1, wg1,
                   bg1, we2, ws2, wk2, wg2, bg2, fc1w, fc1b, hf_ref):
    def dot(a, b):
        return jnp.dot(a, b, preferred_element_type=jnp.float32)

    def dot_exact(a_bf16, x):
        acc = jnp.zeros((a_bf16.shape[0], x.shape[1]), jnp.float32)
        r = x
        for _ in range(3):
            c = r.astype(jnp.bfloat16)
            r = r - c.astype(jnp.float32)
            acc += jnp.dot(a_bf16, c, preferred_element_type=jnp.float32)
        return acc

    asum = a_ref[0] + a_ref[1]
    a_bf16 = jnp.concatenate([asum[t] for t in range(T)],
                             axis=1).astype(jnp.bfloat16)

    def layer(h, we, ws, wk, wg, bg, fin):
        hw = jnp.concatenate([dot(h, we[t]) for t in range(T)], axis=0)
        agg = dot_exact(a_bf16, hw)
        u = agg + dot(h, ws[...])
        g = jax.nn.sigmoid(dot(h, wg[:fin, :]) + dot(u, wg[fin:, :])
                           + bg[...][None, :])
        return g * _leaky(u) + (1.0 - g) * dot(h, wk[...])

    x = x_ref[...]
    h = layer(x, we0, ws0, wk0, wg0, bg0, IN_FEATS)
    h = layer(h, we1, ws1, wk1, wg1, bg1, HID)
    h = layer(h, we2, ws2, wk2, wg2, bg2, HID)
    feat = _leaky(dot(x, fc1w[...]) + fc1b[...][None, :])
    hf_ref[...] = jnp.concatenate([h, feat], axis=1)


def _run_layers(x, a, args):
    return pl.pallas_call(
        _layers_kernel,
        out_shape=jax.ShapeDtypeStruct((N, 2 * HID), jnp.float32),
    )(x, a, *args)


def _fc2_kernel(flat_ref, w_ref, b_ref, out_ref):
    i = pl.program_id(0)

    @pl.when(i == 0)
    def _():
        out_ref[...] = jnp.zeros_like(out_ref)

    out_ref[...] += jnp.dot(flat_ref[...], w_ref[...],
                            preferred_element_type=jnp.float32)

    @pl.when(i == NUM_FC2_BLKS - 1)
    def _():
        out_ref[...] = _leaky(out_ref[...] + b_ref[...])


def _run_fc2(flat, w, b):
    return pl.pallas_call(
        _fc2_kernel,
        grid=(NUM_FC2_BLKS,),
        in_specs=[
            pl.BlockSpec((1, FC2_BLK), lambda i: (0, i)),
            pl.BlockSpec((FC2_BLK, OUT_DIM), lambda i: (i, 0)),
            pl.BlockSpec((1, OUT_DIM), lambda i: (0, 0)),
        ],
        out_specs=pl.BlockSpec((1, OUT_DIM), lambda i: (0, 0)),
        out_shape=jax.ShapeDtypeStruct((1, OUT_DIM), jnp.float32),
    )(flat, w, b)


def kernel(x, edge_index, edge_type, W_et0, W_self0, W_skip0, W_gate0,
           b_gate0, W_et1, W_self1, W_skip1, W_gate1, b_gate1, W_et2,
           W_self2, W_skip2, W_gate2, b_gate2, fc1_W, fc1_b, fc2_W, fc2_b):
    b_parts = _build_b_sc(edge_index, edge_type)
    a = b_parts.reshape(NSC, T, N, N)
    hf = _run_layers(x, a, (W_et0, W_self0, W_skip0, W_gate0, b_gate0,
                            W_et1, W_self1, W_skip1, W_gate1, b_gate1,
                            W_et2, W_self2, W_skip2, W_gate2, b_gate2,
                            fc1_W, fc1_b))
    flat = hf.reshape(1, N * 2 * HID)
    return _run_fc2(flat, fc2_W, fc2_b.reshape(1, OUT_DIM))
